# baseline (device time: 29063 ns/iter reference)
import jax
import jax.numpy as jnp
from jax import lax
from jax.experimental import pallas as pl
from jax.experimental.pallas import tpu as pltpu

N_DEV = 8


def kernel(q, k, v):
    s_per, d = q.shape
    scale = 1.0 / (d ** 0.5)

    def body(q_ref, k_ref, v_ref, out_ref, kv_ref, send_sems, recv_sems):
        my_pos = lax.axis_index("i")
        left = lax.rem(my_pos - 1 + N_DEV, N_DEV)
        right = lax.rem(my_pos + 1, N_DEV)

        barrier_sem = pltpu.get_barrier_semaphore()
        for nbr in (left, right):
            pl.semaphore_signal(
                barrier_sem,
                inc=1,
                device_id=(nbr,),
                device_id_type=pl.DeviceIdType.MESH,
            )
        pl.semaphore_wait(barrier_sem, 2)

        q_bf = q_ref[:, :].astype(jnp.bfloat16)
        kv_ref[0, :s_per, :] = k_ref[:, :].astype(jnp.bfloat16)
        kv_ref[0, s_per:, :] = v_ref[:, :].astype(jnp.bfloat16)

        m = jnp.full((s_per, 1), -jnp.inf, dtype=jnp.float32)
        l = jnp.zeros((s_per, 1), dtype=jnp.float32)
        acc = jnp.zeros((s_per, d), dtype=jnp.float32)

        for h in range(N_DEV):
            if h < N_DEV - 1:
                rdma = pltpu.make_async_remote_copy(
                    src_ref=kv_ref.at[h],
                    dst_ref=kv_ref.at[h + 1],
                    send_sem=send_sems.at[h],
                    recv_sem=recv_sems.at[h + 1],
                    device_id=(right,),
                    device_id_type=pl.DeviceIdType.MESH,
                )
                rdma.start()

            k_chunk = kv_ref[h, :s_per, :]
            v_chunk = kv_ref[h, s_per:, :]
            s = (
                lax.dot_general(
                    q_bf,
                    k_chunk,
                    (((1,), (1,)), ((), ())),
                    preferred_element_type=jnp.float32,
                )
                * scale
            )
            m_new = jnp.maximum(m, jnp.max(s, axis=1, keepdims=True))
            alpha = jnp.exp(m - m_new)
            p = jnp.exp(s - m_new)
            acc = acc * alpha + lax.dot_general(
                p.astype(jnp.bfloat16),
                v_chunk,
                (((1,), (0,)), ((), ())),
                preferred_element_type=jnp.float32,
            )
            l = l * alpha + jnp.sum(p, axis=1, keepdims=True)
            m = m_new

            if h < N_DEV - 1:
                rdma.wait()

        out_ref[:, :] = acc / l

    return pl.pallas_call(
        body,
        out_shape=jax.ShapeDtypeStruct((s_per, d), jnp.float32),
        in_specs=[pl.BlockSpec(memory_space=pltpu.VMEM)] * 3,
        out_specs=pl.BlockSpec(memory_space=pltpu.VMEM),
        scratch_shapes=[
            pltpu.VMEM((N_DEV, 2 * s_per, d), jnp.bfloat16),
            pltpu.SemaphoreType.DMA((N_DEV,)),
            pltpu.SemaphoreType.DMA((N_DEV,)),
        ],
        compiler_params=pltpu.CompilerParams(collective_id=0),
    )(q, k, v)


# device time: 16126 ns/iter; 1.8022x vs baseline; 1.8022x over previous
import jax
import jax.numpy as jnp
from jax import lax
from jax.experimental import pallas as pl
from jax.experimental.pallas import tpu as pltpu

N_DEV = 8


def kernel(q, k, v):
    s_per, d = q.shape
    scale = 1.0 / (d ** 0.5)

    def body(
        q_ref,
        k_ref,
        v_ref,
        out_ref,
        k_all,
        v_all,
        send_sems_k,
        send_sems_v,
        recv_sems_k,
        recv_sems_v,
    ):
        my_pos = lax.axis_index("i")

        barrier_sem = pltpu.get_barrier_semaphore()
        for j in range(1, N_DEV):
            pl.semaphore_signal(
                barrier_sem,
                inc=1,
                device_id=(lax.rem(my_pos + j, N_DEV),),
                device_id_type=pl.DeviceIdType.MESH,
            )
        pl.semaphore_wait(barrier_sem, N_DEV - 1)

        k_all[pl.ds(my_pos * s_per, s_per), :] = k_ref[:, :].astype(jnp.bfloat16)
        v_all[pl.ds(my_pos * s_per, s_per), :] = v_ref[:, :].astype(jnp.bfloat16)

        sends = []
        for j in range(1, N_DEV):
            dst = lax.rem(my_pos + j, N_DEV)
            for all_ref, ssems, rsems in (
                (k_all, send_sems_k, recv_sems_k),
                (v_all, send_sems_v, recv_sems_v),
            ):
                rdma = pltpu.make_async_remote_copy(
                    src_ref=all_ref.at[pl.ds(my_pos * s_per, s_per)],
                    dst_ref=all_ref.at[pl.ds(my_pos * s_per, s_per)],
                    send_sem=ssems.at[j],
                    recv_sem=rsems.at[my_pos],
                    device_id=(dst,),
                    device_id_type=pl.DeviceIdType.MESH,
                )
                rdma.start()
                sends.append(rdma)

        q_bf = q_ref[:, :].astype(jnp.bfloat16)

        for j in range(1, N_DEV):
            o = lax.rem(my_pos + j, N_DEV)
            for all_ref, ssems, rsems in (
                (k_all, send_sems_k, recv_sems_k),
                (v_all, send_sems_v, recv_sems_v),
            ):
                recv = pltpu.make_async_remote_copy(
                    src_ref=all_ref.at[pl.ds(o * s_per, s_per)],
                    dst_ref=all_ref.at[pl.ds(o * s_per, s_per)],
                    send_sem=ssems.at[j],
                    recv_sem=rsems.at[o],
                    device_id=(o,),
                    device_id_type=pl.DeviceIdType.MESH,
                )
                recv.wait_recv()

        s = (
            lax.dot_general(
                q_bf,
                k_all[:, :],
                (((1,), (1,)), ((), ())),
                preferred_element_type=jnp.float32,
            )
            * scale
        )
        m = jnp.max(s, axis=1, keepdims=True)
        p = jnp.exp(s - m)
        l = jnp.sum(p, axis=1, keepdims=True)
        pv = lax.dot_general(
            p.astype(jnp.bfloat16),
            v_all[:, :],
            (((1,), (0,)), ((), ())),
            preferred_element_type=jnp.float32,
        )
        out_ref[:, :] = pv / l

        for rdma in sends:
            rdma.wait_send()

    return pl.pallas_call(
        body,
        out_shape=jax.ShapeDtypeStruct((s_per, d), jnp.float32),
        in_specs=[pl.BlockSpec(memory_space=pltpu.VMEM)] * 3,
        out_specs=pl.BlockSpec(memory_space=pltpu.VMEM),
        scratch_shapes=[
            pltpu.VMEM((N_DEV * s_per, d), jnp.bfloat16),
            pltpu.VMEM((N_DEV * s_per, d), jnp.bfloat16),
            pltpu.SemaphoreType.DMA((N_DEV,)),
            pltpu.SemaphoreType.DMA((N_DEV,)),
            pltpu.SemaphoreType.DMA((N_DEV,)),
            pltpu.SemaphoreType.DMA((N_DEV,)),
        ],
        compiler_params=pltpu.CompilerParams(collective_id=0),
    )(q, k, v)
